# baseline (speedup 1.0000x reference)
import functools

import jax
import jax.numpy as jnp
from jax import lax
from jax.experimental import pallas as pl
from jax.experimental.pallas import tpu as pltpu
from jax.experimental.pallas import tpu_sc as plsc

_BETA = 25.0
_NC = 2
_NS = 16
_NW = _NC * _NS
_LANES = 16
_SLICES = (128, 192, 192)


def _make_body(B, D, K):
    rows = B // _NW
    assert sum(_SLICES) == rows
    nchunk = K // _LANES
    nsl = len(_SLICES)
    starts = [sum(_SLICES[:i]) for i in range(nsl)]

    def body(x_hbm, pairs_hbm, out_hbm,
             x_v0, x_v1, x_v2, o_v0, o_v1, o_v2, pairs_v,
             si0, si1, si2, so0, so1, so2):
        xbufs = (x_v0, x_v1, x_v2)
        obufs = (o_v0, o_v1, o_v2)
        sin = (si0, si1, si2)
        sout = (so0, so1, so2)

        wid = lax.axis_index("s") * _NC + lax.axis_index("c")
        base = wid * rows

        in_h = []
        for h in range(nsl):
            src = x_hbm.at[pl.ds((base + starts[h]) * D, _SLICES[h] * D)]
            dst = xbufs[h].at[pl.ds(0, _SLICES[h] * D)]
            in_h.append(pltpu.async_copy(src, dst, sin[h]))

        pltpu.sync_copy(pairs_hbm, pairs_v)
        lane = lax.iota(jnp.int32, _LANES)
        idx_i = []
        idx_j = []
        for c in range(nchunk):
            kvec = (c * _LANES + lane) * 2
            idx_i.append(plsc.load_gather(pairs_v, [kvec]))
            idx_j.append(plsc.load_gather(pairs_v, [kvec + 1]))

        out_h = []
        for h in range(nsl):
            in_h[h].wait()
            x_v = xbufs[h]
            o_v = obufs[h]

            @plsc.parallel_loop(0, _SLICES[h], 1, unroll=8)
            def _row(r):
                xrow = x_v.at[pl.ds(r * D, D)]
                orow = o_v.at[pl.ds(r * K, K)]
                for c in range(nchunk):
                    xi = plsc.load_gather(xrow, [idx_i[c]])
                    xj = plsc.load_gather(xrow, [idx_j[c]])
                    z = (xj - xi) * _BETA
                    orow[pl.ds(c * _LANES, _LANES)] = 1.0 / (1.0 + jnp.exp(z))

            src = o_v.at[pl.ds(0, _SLICES[h] * K)]
            dst = out_hbm.at[pl.ds((base + starts[h]) * K, _SLICES[h] * K)]
            out_h.append(pltpu.async_copy(src, dst, sout[h]))

        for h in out_h:
            h.wait()

    return body


def kernel(x, pairs):
    B, D = x.shape
    K = pairs.shape[0]
    smax = max(_SLICES)
    run = pl.kernel(
        _make_body(B, D, K),
        out_type=jax.ShapeDtypeStruct((B * K,), jnp.float32),
        mesh=plsc.VectorSubcoreMesh(core_axis_name="c", subcore_axis_name="s"),
        compiler_params=pltpu.CompilerParams(needs_layout_passes=False),
        scratch_types=(
            [pltpu.VMEM((smax * D,), jnp.float32)] * len(_SLICES)
            + [pltpu.VMEM((smax * K,), jnp.float32)] * len(_SLICES)
            + [pltpu.VMEM((K * 2,), jnp.int32)]
            + [pltpu.SemaphoreType.DMA] * (2 * len(_SLICES))
        ),
    )
    out = run(x.reshape(B * D), pairs.reshape(K * 2))
    return out.reshape(B, K)


# speedup vs baseline: 1.0245x; 1.0245x over previous
import functools

import jax
import jax.numpy as jnp
from jax import lax
from jax.experimental import pallas as pl
from jax.experimental.pallas import tpu as pltpu
from jax.experimental.pallas import tpu_sc as plsc

_BETA = 25.0
_NC = 2
_NS = 16
_NW = _NC * _NS
_LANES = 16
_NHALF = 2


def _make_body(B, D, K):
    rows = B // _NW
    half = rows // _NHALF
    nchunk = K // _LANES

    def body(x_hbm, pairs_hbm, out_hbm,
             x_v0, x_v1, o_v0, o_v1, pairs_v,
             si0, si1, so0, so1):
        xbufs = (x_v0, x_v1)
        obufs = (o_v0, o_v1)
        sin = (si0, si1)
        sout = (so0, so1)

        wid = lax.axis_index("s") * _NC + lax.axis_index("c")
        base = wid * rows

        in_h = []
        for h in range(_NHALF):
            src = x_hbm.at[pl.ds((base + h * half) * D, half * D)]
            in_h.append(pltpu.async_copy(src, xbufs[h], sin[h]))

        pltpu.sync_copy(pairs_hbm, pairs_v)
        lane = lax.iota(jnp.int32, _LANES)
        idx_i = []
        idx_j = []
        for c in range(nchunk):
            kvec = (c * _LANES + lane) * 2
            idx_i.append(plsc.load_gather(pairs_v, [kvec]))
            idx_j.append(plsc.load_gather(pairs_v, [kvec + 1]))

        out_h = []
        for h in range(_NHALF):
            in_h[h].wait()
            x_v = xbufs[h]
            o_v = obufs[h]

            @plsc.parallel_loop(0, half, 1, unroll=8)
            def _row(r):
                xrow = x_v.at[pl.ds(r * D, D)]
                orow = o_v.at[pl.ds(r * K, K)]
                for c in range(nchunk):
                    xi = plsc.load_gather(xrow, [idx_i[c]])
                    xj = plsc.load_gather(xrow, [idx_j[c]])
                    z = (xj - xi) * _BETA
                    orow[pl.ds(c * _LANES, _LANES)] = 1.0 / (1.0 + jnp.exp(z))

            dst = out_hbm.at[pl.ds((base + h * half) * K, half * K)]
            out_h.append(pltpu.async_copy(o_v, dst, sout[h]))

        for h in out_h:
            h.wait()

    return body


def kernel(x, pairs):
    B, D = x.shape
    K = pairs.shape[0]
    rows = B // _NW
    half = rows // _NHALF
    run = pl.kernel(
        _make_body(B, D, K),
        out_type=jax.ShapeDtypeStruct((B * K,), jnp.float32),
        mesh=plsc.VectorSubcoreMesh(core_axis_name="c", subcore_axis_name="s"),
        compiler_params=pltpu.CompilerParams(needs_layout_passes=False),
        scratch_types=[
            pltpu.VMEM((half * D,), jnp.float32),
            pltpu.VMEM((half * D,), jnp.float32),
            pltpu.VMEM((half * K,), jnp.float32),
            pltpu.VMEM((half * K,), jnp.float32),
            pltpu.VMEM((K * 2,), jnp.int32),
            pltpu.SemaphoreType.DMA,
            pltpu.SemaphoreType.DMA,
            pltpu.SemaphoreType.DMA,
            pltpu.SemaphoreType.DMA,
        ],
    )
    out = run(x.reshape(B * D), pairs.reshape(K * 2))
    return out.reshape(B, K)
